# baseline (speedup 1.0000x reference)
import functools

import jax
import jax.numpy as jnp
from jax import lax
from jax.experimental import pallas as pl
from jax.experimental.pallas import tpu as pltpu
from jax.experimental.pallas import tpu_sc as plsc

KSEG = 16
NC, NS = 2, 16
NW = NC * NS
NPIX = 32 * 512 * 512
ROWS_C = 8
PIX_CHUNK = ROWS_C * 512
NCHUNK_TOT = NPIX // PIX_CHUNK
NCHUNK = NCHUNK_TOT // NW
BLK_PER_IMG = 512 // ROWS_C


def _isp_body(x_hbm, params_hbm, slopes_hbm, out_hbm,
              inbuf, outbuf, params_v, slopes_v,
              a0, a1, a2, b0, b1, b2):
    f32 = jnp.float32
    wid = lax.axis_index("s") * NC + lax.axis_index("c")
    cbase = wid * NCHUNK

    pltpu.sync_copy(params_hbm, params_v)
    pltpu.sync_copy(slopes_hbm, slopes_v)

    iota_i = lax.iota(jnp.int32, KSEG)
    knots = iota_i.astype(f32) * (1.0 / KSEG)

    for c, (at, bt) in enumerate(((a0, b0), (a1, b1), (a2, b2))):
        r = slopes_v[pl.ds(c * KSEG, KSEG)]
        e = jnp.exp(r - jnp.max(r))
        s_vec = jnp.broadcast_to(jnp.sum(e), (KSEG,))
        slope = e * (KSEG / s_vec)
        h = slope * (1.0 / KSEG)
        cum_ex = plsc.cumsum(h) - h
        at[...] = slope
        bt[...] = cum_ex - slope * knots

    pv = params_v[...]
    coef = []
    for i in range(3):
        t_i = pv[9 + i]
        coef.append(tuple(t_i * pv[3 * i + j] for j in range(3)))
    bias = tuple(pv[12 + i] for i in range(3))

    col0 = iota_i * 0
    col1 = col0 + 1
    col2 = col0 + 2

    def chunk_step(k, _):
        cid = cbase + k
        img = cid // BLK_PER_IMG
        r0 = (cid % BLK_PER_IMG) * ROWS_C
        pltpu.sync_copy(x_hbm.at[img, pl.ds(r0, ROWS_C)], inbuf)

        @plsc.parallel_loop(0, PIX_CHUNK, step=KSEG, unroll=4)
        def body(base):
            rowv = jnp.broadcast_to(base >> 9, (KSEG,))
            colv = iota_i + (base & 511)
            r = plsc.load_gather(inbuf, [rowv, colv, col0])
            g = plsc.load_gather(inbuf, [rowv, colv, col1])
            bl = plsc.load_gather(inbuf, [rowv, colv, col2])
            for c, (at, bt, cc) in enumerate(((a0, b0, col0),
                                             (a1, b1, col1),
                                             (a2, b2, col2))):
                y = coef[c][0] * r + coef[c][1] * g + coef[c][2] * bl + bias[c]
                y = jnp.clip(y, 0.0, 1.0)
                idx = jnp.minimum((y * KSEG).astype(jnp.int32), KSEG - 1)
                av = plsc.load_gather(at, [idx])
                bv = plsc.load_gather(bt, [idx])
                f = jnp.clip(av * y + bv, 0.0, 1.0)
                plsc.store_scatter(outbuf, [rowv, colv, cc], f)

        pltpu.sync_copy(outbuf, out_hbm.at[img, pl.ds(r0, ROWS_C)])
        return 0

    lax.fori_loop(0, NCHUNK, chunk_step, 0)


_mesh = plsc.VectorSubcoreMesh(core_axis_name="c", subcore_axis_name="s",
                               num_cores=NC, num_subcores=NS)

_isp = functools.partial(
    pl.kernel,
    out_type=jax.ShapeDtypeStruct((32, 512, 512, 3), jnp.float32),
    mesh=_mesh,
    compiler_params=pltpu.CompilerParams(needs_layout_passes=False, use_tc_tiling_on_sc=False),
    scratch_types=[
        pltpu.VMEM((ROWS_C, 512, 3), jnp.float32),
        pltpu.VMEM((ROWS_C, 512, 3), jnp.float32),
        pltpu.VMEM((16,), jnp.float32),
        pltpu.VMEM((48,), jnp.float32),
        pltpu.VMEM((KSEG,), jnp.float32),
        pltpu.VMEM((KSEG,), jnp.float32),
        pltpu.VMEM((KSEG,), jnp.float32),
        pltpu.VMEM((KSEG,), jnp.float32),
        pltpu.VMEM((KSEG,), jnp.float32),
        pltpu.VMEM((KSEG,), jnp.float32),
    ],
)(_isp_body)


@jax.jit
def kernel(x, M, T, b, raw_slopes):
    params = jnp.concatenate(
        [M.reshape(-1), T, b, jnp.zeros((1,), jnp.float32)])
    slopes_t = raw_slopes.T.reshape(-1)
    return _isp(x, params, slopes_t)


# speedup vs baseline: 54.6633x; 54.6633x over previous
import functools

import jax
import jax.numpy as jnp
from jax import lax
from jax.experimental import pallas as pl
from jax.experimental.pallas import tpu as pltpu
from jax.experimental.pallas import tpu_sc as plsc

KSEG = 16
NC, NS = 2, 16
NW = NC * NS
NIMG, NROW, NCOL = 32, 512, 512
ROWS_C = 16
PIX_CHUNK = ROWS_C * NCOL
BLK_PER_IMG = NROW // ROWS_C
NCHUNK_TOT = NIMG * BLK_PER_IMG
NCHUNK = NCHUNK_TOT // NW


def _isp_body(x_hbm, params_hbm, slopes_hbm, out_hbm,
              in0, in1, in2, o0, o1, o2, params_v, slopes_v,
              a0, a1, a2, b0, b1, b2):
    f32 = jnp.float32
    wid = lax.axis_index("s") * NC + lax.axis_index("c")
    cbase = wid * NCHUNK

    pltpu.sync_copy(params_hbm, params_v)
    pltpu.sync_copy(slopes_hbm, slopes_v)

    iota_i = lax.iota(jnp.int32, KSEG)
    knots = iota_i.astype(f32) * (1.0 / KSEG)

    for c, (at, bt) in enumerate(((a0, b0), (a1, b1), (a2, b2))):
        r = slopes_v[pl.ds(c * KSEG, KSEG)]
        e = jnp.exp(r - jnp.max(r))
        s_vec = jnp.broadcast_to(jnp.sum(e), (KSEG,))
        slope = e * (KSEG / s_vec)
        h = slope * (1.0 / KSEG)
        cum_ex = plsc.cumsum(h) - h
        at[...] = slope
        bt[...] = cum_ex - slope * knots

    pv = params_v[...]
    coef = []
    for i in range(3):
        t_i = pv[9 + i]
        coef.append(tuple(t_i * pv[3 * i + j] for j in range(3)))
    bias = tuple(pv[12 + i] for i in range(3))

    def chunk_step(k, _):
        cid = cbase + k
        img = cid // BLK_PER_IMG
        r0 = (cid % BLK_PER_IMG) * ROWS_C
        pltpu.sync_copy(x_hbm.at[img, 0, pl.ds(r0, ROWS_C)], in0)
        pltpu.sync_copy(x_hbm.at[img, 1, pl.ds(r0, ROWS_C)], in1)
        pltpu.sync_copy(x_hbm.at[img, 2, pl.ds(r0, ROWS_C)], in2)

        @plsc.parallel_loop(0, PIX_CHUNK, step=KSEG, unroll=4)
        def body(p):
            row = p >> 9
            col = p & (NCOL - 1)
            r = in0[row, pl.ds(col, KSEG)]
            g = in1[row, pl.ds(col, KSEG)]
            bl = in2[row, pl.ds(col, KSEG)]
            for c, (at, bt, ob) in enumerate(((a0, b0, o0),
                                             (a1, b1, o1),
                                             (a2, b2, o2))):
                y = coef[c][0] * r + coef[c][1] * g + coef[c][2] * bl + bias[c]
                y = jnp.clip(y, 0.0, 1.0)
                idx = jnp.minimum((y * KSEG).astype(jnp.int32), KSEG - 1)
                av = plsc.load_gather(at, [idx])
                bv = plsc.load_gather(bt, [idx])
                f = jnp.clip(av * y + bv, 0.0, 1.0)
                ob[row, pl.ds(col, KSEG)] = f

        pltpu.sync_copy(o0, out_hbm.at[img, 0, pl.ds(r0, ROWS_C)])
        pltpu.sync_copy(o1, out_hbm.at[img, 1, pl.ds(r0, ROWS_C)])
        pltpu.sync_copy(o2, out_hbm.at[img, 2, pl.ds(r0, ROWS_C)])
        return 0

    lax.fori_loop(0, NCHUNK, chunk_step, 0)


_mesh = plsc.VectorSubcoreMesh(core_axis_name="c", subcore_axis_name="s",
                               num_cores=NC, num_subcores=NS)

_isp = functools.partial(
    pl.kernel,
    out_type=jax.ShapeDtypeStruct((NIMG, 3, NROW, NCOL), jnp.float32),
    mesh=_mesh,
    compiler_params=pltpu.CompilerParams(needs_layout_passes=False),
    scratch_types=[
        pltpu.VMEM((ROWS_C, NCOL), jnp.float32),
        pltpu.VMEM((ROWS_C, NCOL), jnp.float32),
        pltpu.VMEM((ROWS_C, NCOL), jnp.float32),
        pltpu.VMEM((ROWS_C, NCOL), jnp.float32),
        pltpu.VMEM((ROWS_C, NCOL), jnp.float32),
        pltpu.VMEM((ROWS_C, NCOL), jnp.float32),
        pltpu.VMEM((16,), jnp.float32),
        pltpu.VMEM((48,), jnp.float32),
        pltpu.VMEM((KSEG,), jnp.float32),
        pltpu.VMEM((KSEG,), jnp.float32),
        pltpu.VMEM((KSEG,), jnp.float32),
        pltpu.VMEM((KSEG,), jnp.float32),
        pltpu.VMEM((KSEG,), jnp.float32),
        pltpu.VMEM((KSEG,), jnp.float32),
    ],
)(_isp_body)


@jax.jit
def kernel(x, M, T, b, raw_slopes):
    params = jnp.concatenate(
        [M.reshape(-1), T, b, jnp.zeros((1,), jnp.float32)])
    slopes_t = raw_slopes.T.reshape(-1)
    xp = jnp.transpose(x, (0, 3, 1, 2))
    out = _isp(xp, params, slopes_t)
    return jnp.transpose(out, (0, 2, 3, 1))


# speedup vs baseline: 111.9065x; 2.0472x over previous
import functools

import jax
import jax.numpy as jnp
from jax import lax
from jax.experimental import pallas as pl
from jax.experimental.pallas import tpu as pltpu
from jax.experimental.pallas import tpu_sc as plsc

KSEG = 16
NC, NS = 2, 16
NW = NC * NS
NIMG, NROW, NCOL = 32, 512, 512
ROWS_C = 16
PIX_CHUNK = ROWS_C * NCOL
BLK_PER_IMG = NROW // ROWS_C
NCHUNK_TOT = NIMG * BLK_PER_IMG
NCHUNK = NCHUNK_TOT // NW
Y16_MAX = 15.999999046325684


def _isp_body(x_hbm, params_hbm, slopes_hbm, out_hbm,
              ia0, ia1, ia2, ib0, ib1, ib2,
              oa0, oa1, oa2, ob0, ob1, ob2,
              params_v, slopes_v,
              a0, a1, a2, b0, b1, b2,
              si0, si1, so0, so1):
    f32 = jnp.float32
    wid = lax.axis_index("s") * NC + lax.axis_index("c")
    cbase = wid * NCHUNK

    pltpu.sync_copy(params_hbm, params_v)
    pltpu.sync_copy(slopes_hbm, slopes_v)

    iota_i = lax.iota(jnp.int32, KSEG)
    knots = iota_i.astype(f32)

    for c, (at, bt) in enumerate(((a0, b0), (a1, b1), (a2, b2))):
        r = slopes_v[pl.ds(c * KSEG, KSEG)]
        e = jnp.exp(r - jnp.max(r))
        s_vec = jnp.broadcast_to(jnp.sum(e), (KSEG,))
        slope = e * (KSEG / s_vec)
        h = slope * (1.0 / KSEG)
        cum_ex = plsc.cumsum(h) - h
        a_tab = slope * (1.0 / KSEG)
        at[...] = a_tab
        bt[...] = cum_ex - a_tab * knots

    pv = params_v[...]
    coef = []
    for i in range(3):
        t16 = pv[9 + i] * float(KSEG)
        coef.append(tuple(t16 * pv[3 * i + j] for j in range(3)))
    bias = tuple(pv[12 + i] * float(KSEG) for i in range(3))

    ibufs = ((ia0, ia1, ia2), (ib0, ib1, ib2))
    obufs = ((oa0, oa1, oa2), (ob0, ob1, ob2))
    isems = (si0, si1)
    osems = (so0, so1)

    def in_slices(k):
        img = k // BLK_PER_IMG
        r0 = (k % BLK_PER_IMG) * ROWS_C
        return [x_hbm.at[img, c, pl.ds(r0, ROWS_C)] for c in range(3)]

    def out_slices(k):
        img = k // BLK_PER_IMG
        r0 = (k % BLK_PER_IMG) * ROWS_C
        return [out_hbm.at[img, c, pl.ds(r0, ROWS_C)] for c in range(3)]

    def start_in(k, par):
        for src, dst in zip(in_slices(k), ibufs[par]):
            pltpu.async_copy(src, dst, isems[par])

    def wait_in(par):
        for src, dst in zip(in_slices(cbase), ibufs[par]):
            pltpu.make_async_copy(src, dst, isems[par]).wait()

    def start_out(k, par):
        for src, dst in zip(obufs[par], out_slices(k)):
            pltpu.async_copy(src, dst, osems[par])

    def wait_out(par):
        for src, dst in zip(obufs[par], out_slices(cbase)):
            pltpu.make_async_copy(src, dst, osems[par]).wait()

    def compute(par):
        in0, in1, in2 = ibufs[par]
        o0, o1, o2 = obufs[par]

        @plsc.parallel_loop(0, PIX_CHUNK, step=KSEG, unroll=4)
        def body(p):
            row = p >> 9
            col = p & (NCOL - 1)
            r = in0[row, pl.ds(col, KSEG)]
            g = in1[row, pl.ds(col, KSEG)]
            bl = in2[row, pl.ds(col, KSEG)]
            for c, (at, bt, ob) in enumerate(((a0, b0, o0),
                                             (a1, b1, o1),
                                             (a2, b2, o2))):
                y16 = (coef[c][0] * r + coef[c][1] * g + coef[c][2] * bl
                       + bias[c])
                y16 = jnp.clip(y16, 0.0, Y16_MAX)
                idx = y16.astype(jnp.int32)
                av = plsc.load_gather(at, [idx])
                bv = plsc.load_gather(bt, [idx])
                ob[row, pl.ds(col, KSEG)] = av * y16 + bv

    start_in(cbase, 0)
    start_in(cbase + 1, 1)

    def step(t, _):
        for par in range(2):
            k = 2 * t + par
            wait_in(par)

            @pl.when(k >= 2)
            def _():
                wait_out(par)

            compute(par)
            start_out(cbase + k, par)

            @pl.when(k + 2 < NCHUNK)
            def _():
                start_in(cbase + k + 2, par)
        return 0

    lax.fori_loop(0, NCHUNK // 2, step, 0)
    wait_out(0)
    wait_out(1)


_mesh = plsc.VectorSubcoreMesh(core_axis_name="c", subcore_axis_name="s",
                               num_cores=NC, num_subcores=NS)

_PLANE = pltpu.VMEM((ROWS_C, NCOL), jnp.float32)

_isp = functools.partial(
    pl.kernel,
    out_type=jax.ShapeDtypeStruct((NIMG, 3, NROW, NCOL), jnp.float32),
    mesh=_mesh,
    compiler_params=pltpu.CompilerParams(needs_layout_passes=False),
    scratch_types=[
        _PLANE, _PLANE, _PLANE,
        _PLANE, _PLANE, _PLANE,
        _PLANE, _PLANE, _PLANE,
        _PLANE, _PLANE, _PLANE,
        pltpu.VMEM((16,), jnp.float32),
        pltpu.VMEM((48,), jnp.float32),
        pltpu.VMEM((KSEG,), jnp.float32),
        pltpu.VMEM((KSEG,), jnp.float32),
        pltpu.VMEM((KSEG,), jnp.float32),
        pltpu.VMEM((KSEG,), jnp.float32),
        pltpu.VMEM((KSEG,), jnp.float32),
        pltpu.VMEM((KSEG,), jnp.float32),
        pltpu.SemaphoreType.DMA,
        pltpu.SemaphoreType.DMA,
        pltpu.SemaphoreType.DMA,
        pltpu.SemaphoreType.DMA,
    ],
)(_isp_body)


@jax.jit
def kernel(x, M, T, b, raw_slopes):
    params = jnp.concatenate(
        [M.reshape(-1), T, b, jnp.zeros((1,), jnp.float32)])
    slopes_t = raw_slopes.T.reshape(-1)
    xp = jnp.transpose(x, (0, 3, 1, 2))
    out = _isp(xp, params, slopes_t)
    return jnp.transpose(out, (0, 2, 3, 1))


# speedup vs baseline: 112.3294x; 1.0038x over previous
import functools

import jax
import jax.numpy as jnp
from jax import lax
from jax.experimental import pallas as pl
from jax.experimental.pallas import tpu as pltpu
from jax.experimental.pallas import tpu_sc as plsc

KSEG = 16
NC, NS = 2, 16
NW = NC * NS
NIMG, NROW, NCOL = 32, 512, 512
ROWS_C = 8
PIX_CHUNK = ROWS_C * NCOL
BLK_PER_IMG = NROW // ROWS_C
NCHUNK_TOT = NIMG * BLK_PER_IMG
NCHUNK = NCHUNK_TOT // NW
Y16_MAX = 15.999999046325684


def _isp_body(x_hbm, params_hbm, slopes_hbm, out_hbm,
              ia0, ia1, ia2, ib0, ib1, ib2, ic0, ic1, ic2, id0, id1, id2,
              oa0, oa1, oa2, ob0, ob1, ob2, oc0, oc1, oc2, od0, od1, od2,
              params_v, slopes_v,
              a0, a1, a2, b0, b1, b2,
              si0, si1, si2, si3, so0, so1, so2, so3):
    f32 = jnp.float32
    wid = lax.axis_index("s") * NC + lax.axis_index("c")
    cbase = wid * NCHUNK

    pltpu.sync_copy(params_hbm, params_v)
    pltpu.sync_copy(slopes_hbm, slopes_v)

    iota_i = lax.iota(jnp.int32, KSEG)
    knots = iota_i.astype(f32)

    for c, (at, bt) in enumerate(((a0, b0), (a1, b1), (a2, b2))):
        r = slopes_v[pl.ds(c * KSEG, KSEG)]
        e = jnp.exp(r - jnp.max(r))
        s_vec = jnp.broadcast_to(jnp.sum(e), (KSEG,))
        slope = e * (KSEG / s_vec)
        h = slope * (1.0 / KSEG)
        cum_ex = plsc.cumsum(h) - h
        a_tab = slope * (1.0 / KSEG)
        at[...] = a_tab
        bt[...] = cum_ex - a_tab * knots

    pv = params_v[...]
    coef = []
    for i in range(3):
        t16 = pv[9 + i] * float(KSEG)
        coef.append(tuple(t16 * pv[3 * i + j] for j in range(3)))
    bias = tuple(pv[12 + i] * float(KSEG) for i in range(3))

    ibufs = ((ia0, ia1, ia2), (ib0, ib1, ib2), (ic0, ic1, ic2),
             (id0, id1, id2))
    obufs = ((oa0, oa1, oa2), (ob0, ob1, ob2), (oc0, oc1, oc2),
             (od0, od1, od2))
    isems = (si0, si1, si2, si3)
    osems = (so0, so1, so2, so3)

    def in_slices(k):
        img = k // BLK_PER_IMG
        r0 = (k % BLK_PER_IMG) * ROWS_C
        return [x_hbm.at[img, c, pl.ds(r0, ROWS_C)] for c in range(3)]

    def out_slices(k):
        img = k // BLK_PER_IMG
        r0 = (k % BLK_PER_IMG) * ROWS_C
        return [out_hbm.at[img, c, pl.ds(r0, ROWS_C)] for c in range(3)]

    def start_in(k, par):
        for src, dst in zip(in_slices(k), ibufs[par]):
            pltpu.async_copy(src, dst, isems[par])

    def wait_in(par):
        for src, dst in zip(in_slices(cbase), ibufs[par]):
            pltpu.make_async_copy(src, dst, isems[par]).wait()

    def start_out(k, par):
        for src, dst in zip(obufs[par], out_slices(k)):
            pltpu.async_copy(src, dst, osems[par])

    def wait_out(par):
        for src, dst in zip(obufs[par], out_slices(cbase)):
            pltpu.make_async_copy(src, dst, osems[par]).wait()

    def compute(par):
        in0, in1, in2 = ibufs[par]
        o0, o1, o2 = obufs[par]

        @plsc.parallel_loop(0, PIX_CHUNK, step=KSEG, unroll=4)
        def body(p):
            row = p >> 9
            col = p & (NCOL - 1)
            r = in0[row, pl.ds(col, KSEG)]
            g = in1[row, pl.ds(col, KSEG)]
            bl = in2[row, pl.ds(col, KSEG)]
            for c, (at, bt, ob) in enumerate(((a0, b0, o0),
                                             (a1, b1, o1),
                                             (a2, b2, o2))):
                y16 = (coef[c][0] * r + coef[c][1] * g + coef[c][2] * bl
                       + bias[c])
                y16 = jnp.clip(y16, 0.0, Y16_MAX)
                idx = y16.astype(jnp.int32)
                av = plsc.load_gather(at, [idx])
                bv = plsc.load_gather(bt, [idx])
                ob[row, pl.ds(col, KSEG)] = av * y16 + bv

    NB = 4
    start_in(cbase, 0)
    start_in(cbase + 1, 1)
    start_in(cbase + 2, 2)
    start_in(cbase + 3, 3)

    def step(t, _):
        for par in range(NB):
            k = NB * t + par
            wait_in(par)

            @pl.when(k >= NB)
            def _():
                wait_out(par)

            compute(par)
            start_out(cbase + k, par)

            @pl.when(k + NB < NCHUNK)
            def _():
                start_in(cbase + k + NB, par)
        return 0

    lax.fori_loop(0, NCHUNK // NB, step, 0)
    wait_out(0)
    wait_out(1)
    wait_out(2)
    wait_out(3)


_mesh = plsc.VectorSubcoreMesh(core_axis_name="c", subcore_axis_name="s",
                               num_cores=NC, num_subcores=NS)

_PLANE = pltpu.VMEM((ROWS_C, NCOL), jnp.float32)

_isp = functools.partial(
    pl.kernel,
    out_type=jax.ShapeDtypeStruct((NIMG, 3, NROW, NCOL), jnp.float32),
    mesh=_mesh,
    compiler_params=pltpu.CompilerParams(needs_layout_passes=False),
    scratch_types=[
        _PLANE, _PLANE, _PLANE,
        _PLANE, _PLANE, _PLANE,
        _PLANE, _PLANE, _PLANE,
        _PLANE, _PLANE, _PLANE,
        _PLANE, _PLANE, _PLANE,
        _PLANE, _PLANE, _PLANE,
        _PLANE, _PLANE, _PLANE,
        _PLANE, _PLANE, _PLANE,
        pltpu.VMEM((16,), jnp.float32),
        pltpu.VMEM((48,), jnp.float32),
        pltpu.VMEM((KSEG,), jnp.float32),
        pltpu.VMEM((KSEG,), jnp.float32),
        pltpu.VMEM((KSEG,), jnp.float32),
        pltpu.VMEM((KSEG,), jnp.float32),
        pltpu.VMEM((KSEG,), jnp.float32),
        pltpu.VMEM((KSEG,), jnp.float32),
        pltpu.SemaphoreType.DMA,
        pltpu.SemaphoreType.DMA,
        pltpu.SemaphoreType.DMA,
        pltpu.SemaphoreType.DMA,
        pltpu.SemaphoreType.DMA,
        pltpu.SemaphoreType.DMA,
        pltpu.SemaphoreType.DMA,
        pltpu.SemaphoreType.DMA,
    ],
)(_isp_body)


@jax.jit
def kernel(x, M, T, b, raw_slopes):
    params = jnp.concatenate(
        [M.reshape(-1), T, b, jnp.zeros((1,), jnp.float32)])
    slopes_t = raw_slopes.T.reshape(-1)
    xp = jnp.transpose(x, (0, 3, 1, 2))
    out = _isp(xp, params, slopes_t)
    return jnp.transpose(out, (0, 2, 3, 1))
